# Initial kernel scaffold; baseline (speedup 1.0000x reference)
import jax
import jax.numpy as jnp
from jax.experimental import pallas as pl


def kernel(category, sub_category, industry, average_score, client_feedback, total_awards_and_tips, cat_table, sub_table, ind_table, W1, b1, W2, b2):
    raise NotImplementedError("write your pallas kernel here")


# speedup vs baseline: 1.6831x; 1.6831x over previous
import functools

import jax
import jax.numpy as jnp
from jax import lax
from jax.experimental import pallas as pl
from jax.experimental.pallas import tpu as pltpu
from jax.experimental.pallas import tpu_sc as plsc

_B = 16384
_D = 64
_NC, _NS = 2, 16
_NW = _NC * _NS
_BPW = _B // _NW
_CHUNK = 128
_NCH = _BPW // _CHUNK

_BB = 1024


def _gather_body(cat_t, sub_t, ind_t, cat_i, sub_i, ind_i,
                 out_c, out_s, out_i,
                 idx, rows_c, rows_s, rows_i, sem):
    wid = lax.axis_index("s") * _NC + lax.axis_index("c")
    row0 = wid * _NCH

    pltpu.sync_copy(cat_i.at[pl.ds(row0, _NCH)], idx.at[0])
    pltpu.sync_copy(sub_i.at[pl.ds(row0, _NCH)], idx.at[1])
    pltpu.sync_copy(ind_i.at[pl.ds(row0, _NCH)], idx.at[2])

    copies = []
    for t, (tab, rows) in enumerate(
            ((cat_t, rows_c), (sub_t, rows_s), (ind_t, rows_i))):
        for j in range(_NCH):
            copies.append(pltpu.async_copy(
                tab.at[idx.at[t, j]],
                rows.at[pl.ds(j * _CHUNK, _CHUNK)], sem))
    for c in copies:
        c.wait()

    base = wid * _BPW
    pltpu.sync_copy(rows_c, out_c.at[pl.ds(base, _BPW)])
    pltpu.sync_copy(rows_s, out_s.at[pl.ds(base, _BPW)])
    pltpu.sync_copy(rows_i, out_i.at[pl.ds(base, _BPW)])


@jax.jit
def _sc_gather(cat_table, sub_table, ind_table, cat_i2, sub_i2, ind_i2):
    mesh = plsc.VectorSubcoreMesh(
        core_axis_name="c", subcore_axis_name="s",
        num_cores=_NC, num_subcores=_NS)
    f = pl.kernel(
        _gather_body,
        out_type=[jax.ShapeDtypeStruct((_B, _D), jnp.float32)] * 3,
        mesh=mesh,
        scratch_types=[
            pltpu.VMEM((3, _NCH, _CHUNK), jnp.int32),
            pltpu.VMEM((_BPW, _D), jnp.float32),
            pltpu.VMEM((_BPW, _D), jnp.float32),
            pltpu.VMEM((_BPW, _D), jnp.float32),
            pltpu.SemaphoreType.DMA,
        ],
        compiler_params=pltpu.CompilerParams(use_tc_tiling_on_sc=False),
    )
    return f(cat_table, sub_table, ind_table, cat_i2, sub_i2, ind_i2)


def _mlp_body(cat_ref, sub_ref, ind_ref, s_ref, w1c_ref, w1s_ref, w1i_ref,
              wsc_ref, b1_ref, w2t_ref, b2_ref, out_ref):
    h = (jnp.dot(cat_ref[:], w1c_ref[:], preferred_element_type=jnp.float32)
         + jnp.dot(sub_ref[:], w1s_ref[:], preferred_element_type=jnp.float32)
         + jnp.dot(ind_ref[:], w1i_ref[:], preferred_element_type=jnp.float32)
         + jnp.dot(s_ref[:], wsc_ref[:], preferred_element_type=jnp.float32)
         + b1_ref[:])
    h = jnp.maximum(h, 0.0)
    out_ref[:] = jnp.dot(h, w2t_ref[:],
                         preferred_element_type=jnp.float32) + b2_ref[:]


@jax.jit
def _tc_mlp(cat_e, sub_e, ind_e, s, w1c, w1s, w1i, wsc, b1r, w2t, b2r):
    n_hid = w1c.shape[1]
    grid = (_B // _BB,)
    emb_spec = pl.BlockSpec((_BB, _D), lambda i: (i, 0))
    full = lambda shape: pl.BlockSpec(shape, lambda i: (0, 0))
    return pl.pallas_call(
        _mlp_body,
        grid=grid,
        in_specs=[
            emb_spec, emb_spec, emb_spec,
            pl.BlockSpec((_BB, 8), lambda i: (i, 0)),
            full((_D, n_hid)), full((_D, n_hid)), full((_D, n_hid)),
            full((8, n_hid)), full((1, n_hid)),
            full((n_hid, _D)), full((1, _D)),
        ],
        out_specs=pl.BlockSpec((_BB, _D), lambda i: (i, 0)),
        out_shape=jax.ShapeDtypeStruct((_B, _D), jnp.float32),
    )(cat_e, sub_e, ind_e, s, w1c, w1s, w1i, wsc, b1r, w2t, b2r)


def kernel(category, sub_category, industry, average_score, client_feedback,
           total_awards_and_tips, cat_table, sub_table, ind_table,
           W1, b1, W2, b2):
    cat_i2 = category.reshape(_B // _CHUNK, _CHUNK)
    sub_i2 = sub_category.reshape(_B // _CHUNK, _CHUNK)
    ind_i2 = industry.reshape(_B // _CHUNK, _CHUNK)

    cat_e, sub_e, ind_e = _sc_gather(
        cat_table, sub_table, ind_table, cat_i2, sub_i2, ind_i2)

    w1t = W1.T
    w1c = w1t[0:_D]
    w1s = w1t[_D:2 * _D]
    w1i = w1t[2 * _D:3 * _D]
    wsc = jnp.concatenate(
        [w1t[3 * _D:], jnp.zeros((8 - (w1t.shape[0] - 3 * _D), w1t.shape[1]),
                                 jnp.float32)], axis=0)
    s = jnp.concatenate(
        [average_score, client_feedback, total_awards_and_tips,
         jnp.zeros((_B, 5), jnp.float32)], axis=1)
    return _tc_mlp(cat_e, sub_e, ind_e, s, w1c, w1s, w1i, wsc,
                   b1.reshape(1, -1), W2.T, b2.reshape(1, -1))


# speedup vs baseline: 1.7034x; 1.0121x over previous
import functools

import jax
import jax.numpy as jnp
from jax import lax
from jax.experimental import pallas as pl
from jax.experimental.pallas import tpu as pltpu
from jax.experimental.pallas import tpu_sc as plsc

_B = 16384
_D = 64
_NC, _NS = 2, 16
_NW = _NC * _NS
_BPW = _B // _NW
_CHUNK = 128
_NCH = _BPW // _CHUNK

_CB = 1024
_HB = 512


def _depad_body(t_ref, o_ref):
    t = t_ref[:]
    tt = jnp.transpose(t)
    t3 = tt.reshape(_CB // 16, 16, 64)
    c = jnp.concatenate([t3[:, 0:8, :], t3[:, 8:16, :]], axis=2)
    o_ref[:] = c.reshape(_CB // 2, 128)


@jax.jit
def _depad(t_t):
    n = t_t.shape[1]
    out_rows = pl.cdiv(n, 16) * 8
    return pl.pallas_call(
        _depad_body, grid=(pl.cdiv(n, _CB),),
        in_specs=[pl.BlockSpec((64, _CB), lambda i: (0, i))],
        out_specs=pl.BlockSpec((_CB // 2, 128), lambda i: (i, 0)),
        out_shape=jax.ShapeDtypeStruct((out_rows, 128), jnp.float32),
    )(t_t)


def _pair_index(idx):
    i = idx.astype(jnp.int32)
    return (i & ~jnp.int32(15)) + 2 * (i & 7) + ((i >> 3) & 1)


def _gather_body(cat_t, sub_t, ind_t, cat_i, sub_i, ind_i,
                 out_c, out_s, out_i,
                 idx, rows_c, rows_s, rows_i, sem):
    wid = lax.axis_index("s") * _NC + lax.axis_index("c")
    row0 = wid * _NCH

    pltpu.sync_copy(cat_i.at[pl.ds(row0, _NCH)], idx.at[0])
    pltpu.sync_copy(sub_i.at[pl.ds(row0, _NCH)], idx.at[1])
    pltpu.sync_copy(ind_i.at[pl.ds(row0, _NCH)], idx.at[2])

    copies = []
    for t, (tab, rows) in enumerate(
            ((cat_t, rows_c), (sub_t, rows_s), (ind_t, rows_i))):
        for j in range(_NCH):
            copies.append(pltpu.async_copy(
                tab.at[idx.at[t, j]],
                rows.at[pl.ds(j * _CHUNK, _CHUNK)], sem))
    for c in copies:
        c.wait()

    base = wid * _BPW
    pltpu.sync_copy(rows_c, out_c.at[pl.ds(base, _BPW)])
    pltpu.sync_copy(rows_s, out_s.at[pl.ds(base, _BPW)])
    pltpu.sync_copy(rows_i, out_i.at[pl.ds(base, _BPW)])


@jax.jit
def _sc_gather(cat_table, sub_table, ind_table, cat_i2, sub_i2, ind_i2):
    mesh = plsc.VectorSubcoreMesh(
        core_axis_name="c", subcore_axis_name="s",
        num_cores=_NC, num_subcores=_NS)
    f = pl.kernel(
        _gather_body,
        out_type=[jax.ShapeDtypeStruct((_B, _D), jnp.float32)] * 3,
        mesh=mesh,
        scratch_types=[
            pltpu.VMEM((3, _NCH, _CHUNK), jnp.int32),
            pltpu.VMEM((_BPW, _D), jnp.float32),
            pltpu.VMEM((_BPW, _D), jnp.float32),
            pltpu.VMEM((_BPW, _D), jnp.float32),
            pltpu.SemaphoreType.DMA,
        ],
        compiler_params=pltpu.CompilerParams(use_tc_tiling_on_sc=False),
    )
    return f(cat_table, sub_table, ind_table, cat_i2, sub_i2, ind_i2)


def _mlp_body(c2_ref, s2_ref, i2_ref, sc2_ref, w1c_ref, w1s_ref, w1i_ref,
              wse_ref, wso_ref, b1_ref, w2t_ref, b2_ref, o_ref):
    def dot(a, b):
        return jnp.dot(a, b, preferred_element_type=jnp.float32)

    he = (dot(c2_ref[:, 0:64], w1c_ref[:]) + dot(s2_ref[:, 0:64], w1s_ref[:])
          + dot(i2_ref[:, 0:64], w1i_ref[:]) + dot(sc2_ref[:], wse_ref[:])
          + b1_ref[:])
    ho = (dot(c2_ref[:, 64:128], w1c_ref[:]) + dot(s2_ref[:, 64:128], w1s_ref[:])
          + dot(i2_ref[:, 64:128], w1i_ref[:]) + dot(sc2_ref[:], wso_ref[:])
          + b1_ref[:])
    he = jnp.maximum(he, 0.0)
    ho = jnp.maximum(ho, 0.0)
    o_ref[:] = jnp.concatenate(
        [dot(he, w2t_ref[:]) + b2_ref[:], dot(ho, w2t_ref[:]) + b2_ref[:]],
        axis=1)


@jax.jit
def _tc_mlp(c2, s2, i2, sc2, w1c, w1s, w1i, wse, wso, b1r, w2t, b2r):
    n_hid = w1c.shape[1]
    pair_spec = pl.BlockSpec((_HB, 128), lambda i: (i, 0))
    full = lambda shape: pl.BlockSpec(shape, lambda i: (0, 0))
    return pl.pallas_call(
        _mlp_body,
        grid=(_B // (2 * _HB),),
        in_specs=[
            pair_spec, pair_spec, pair_spec,
            pl.BlockSpec((_HB, 16), lambda i: (i, 0)),
            full((_D, n_hid)), full((_D, n_hid)), full((_D, n_hid)),
            full((16, n_hid)), full((16, n_hid)), full((1, n_hid)),
            full((n_hid, _D)), full((1, _D)),
        ],
        out_specs=pair_spec,
        out_shape=jax.ShapeDtypeStruct((_B // 2, 128), jnp.float32),
    )(c2, s2, i2, sc2, w1c, w1s, w1i, wse, wso, b1r, w2t, b2r)


def kernel(category, sub_category, industry, average_score, client_feedback,
           total_awards_and_tips, cat_table, sub_table, ind_table,
           W1, b1, W2, b2):
    sub2 = _depad(sub_table.T)
    sub_flat = sub2.reshape(2 * sub2.shape[0], _D)

    cat_i2 = category.reshape(_B // _CHUNK, _CHUNK)
    sub_i2 = _pair_index(sub_category).reshape(_B // _CHUNK, _CHUNK)
    ind_i2 = industry.reshape(_B // _CHUNK, _CHUNK)

    cat_e, sub_e, ind_e = _sc_gather(
        cat_table, sub_flat, ind_table, cat_i2, sub_i2, ind_i2)
    c2 = cat_e.reshape(_B // 2, 128)
    s2 = sub_e.reshape(_B // 2, 128)
    i2 = ind_e.reshape(_B // 2, 128)

    w1t = W1.T
    w1c = w1t[0:_D]
    w1s = w1t[_D:2 * _D]
    w1i = w1t[2 * _D:3 * _D]
    nsc = w1t.shape[0] - 3 * _D
    wsc8 = jnp.concatenate(
        [w1t[3 * _D:], jnp.zeros((8 - nsc, w1t.shape[1]), jnp.float32)], axis=0)
    z8 = jnp.zeros_like(wsc8)
    wse = jnp.concatenate([wsc8, z8], axis=0)
    wso = jnp.concatenate([z8, wsc8], axis=0)
    s_all = jnp.concatenate(
        [average_score, client_feedback, total_awards_and_tips,
         jnp.zeros((_B, 5), jnp.float32)], axis=1)
    sc2 = s_all.reshape(_B // 2, 16)

    o2 = _tc_mlp(c2, s2, i2, sc2, w1c, w1s, w1i, wse, wso,
                 b1.reshape(1, -1), W2.T, b2.reshape(1, -1))
    return o2.reshape(_B, _D)


# speedup vs baseline: 1.7541x; 1.0298x over previous
import functools

import jax
import jax.numpy as jnp
from jax import lax
from jax.experimental import pallas as pl
from jax.experimental.pallas import tpu as pltpu
from jax.experimental.pallas import tpu_sc as plsc

_B = 16384
_D = 64
_NC, _NS = 2, 16
_NW = _NC * _NS
_BPW = _B // _NW
_CHUNK = 128
_NCH = _BPW // _CHUNK

_CB = 4096
_HB = 512


def _depad_body(t_ref, o_ref):
    t = t_ref[:]
    tt = jnp.transpose(t)
    t3 = tt.reshape(_CB // 16, 16, 64)
    c = jnp.concatenate([t3[:, 0:8, :], t3[:, 8:16, :]], axis=2)
    o_ref[:] = c.reshape(_CB // 2, 128)


@jax.jit
def _depad(t_t):
    n = t_t.shape[1]
    out_rows = pl.cdiv(n, 16) * 8
    return pl.pallas_call(
        _depad_body, grid=(pl.cdiv(n, _CB),),
        in_specs=[pl.BlockSpec((64, _CB), lambda i: (0, i))],
        out_specs=pl.BlockSpec((_CB // 2, 128), lambda i: (i, 0)),
        out_shape=jax.ShapeDtypeStruct((out_rows, 128), jnp.float32),
    )(t_t)


def _pair_index(idx):
    i = idx.astype(jnp.int32)
    return (i & ~jnp.int32(15)) + 2 * (i & 7) + ((i >> 3) & 1)


def _gather_body(cat_t, sub_t, ind_t, cat_i, sub_i, ind_i,
                 out_c, out_s, out_i,
                 idx, rows_c, rows_s, rows_i, sem):
    wid = lax.axis_index("s") * _NC + lax.axis_index("c")
    row0 = wid * _NCH

    pltpu.sync_copy(cat_i.at[pl.ds(row0, _NCH)], idx.at[0])
    pltpu.sync_copy(sub_i.at[pl.ds(row0, _NCH)], idx.at[1])
    pltpu.sync_copy(ind_i.at[pl.ds(row0, _NCH)], idx.at[2])

    copies = []
    for t, (tab, rows) in enumerate(
            ((cat_t, rows_c), (sub_t, rows_s), (ind_t, rows_i))):
        for j in range(_NCH):
            copies.append(pltpu.async_copy(
                tab.at[idx.at[t, j]],
                rows.at[pl.ds(j * _CHUNK, _CHUNK)], sem))
    for c in copies:
        c.wait()

    base = wid * _BPW
    pltpu.sync_copy(rows_c, out_c.at[pl.ds(base, _BPW)])
    pltpu.sync_copy(rows_s, out_s.at[pl.ds(base, _BPW)])
    pltpu.sync_copy(rows_i, out_i.at[pl.ds(base, _BPW)])


@jax.jit
def _sc_gather(cat_table, sub_table, ind_table, cat_i2, sub_i2, ind_i2):
    mesh = plsc.VectorSubcoreMesh(
        core_axis_name="c", subcore_axis_name="s",
        num_cores=_NC, num_subcores=_NS)
    f = pl.kernel(
        _gather_body,
        out_type=[jax.ShapeDtypeStruct((_B, _D), jnp.float32)] * 3,
        mesh=mesh,
        scratch_types=[
            pltpu.VMEM((3, _NCH, _CHUNK), jnp.int32),
            pltpu.VMEM((_BPW, _D), jnp.float32),
            pltpu.VMEM((_BPW, _D), jnp.float32),
            pltpu.VMEM((_BPW, _D), jnp.float32),
            pltpu.SemaphoreType.DMA,
        ],
        compiler_params=pltpu.CompilerParams(use_tc_tiling_on_sc=False),
    )
    return f(cat_table, sub_table, ind_table, cat_i2, sub_i2, ind_i2)


def _mlp_body(c2_ref, s2_ref, i2_ref, sc2_ref, w1c_ref, w1s_ref, w1i_ref,
              wse_ref, wso_ref, b1_ref, w2t_ref, b2_ref, o_ref):
    def dot(a, b):
        return jnp.dot(a, b, preferred_element_type=jnp.float32)

    he = (dot(c2_ref[:, 0:64], w1c_ref[:]) + dot(s2_ref[:, 0:64], w1s_ref[:])
          + dot(i2_ref[:, 0:64], w1i_ref[:]) + dot(sc2_ref[:], wse_ref[:])
          + b1_ref[:])
    ho = (dot(c2_ref[:, 64:128], w1c_ref[:]) + dot(s2_ref[:, 64:128], w1s_ref[:])
          + dot(i2_ref[:, 64:128], w1i_ref[:]) + dot(sc2_ref[:], wso_ref[:])
          + b1_ref[:])
    he = jnp.maximum(he, 0.0)
    ho = jnp.maximum(ho, 0.0)
    o_ref[:] = jnp.concatenate(
        [dot(he, w2t_ref[:]) + b2_ref[:], dot(ho, w2t_ref[:]) + b2_ref[:]],
        axis=1)


@jax.jit
def _tc_mlp(c2, s2, i2, sc2, w1c, w1s, w1i, wse, wso, b1r, w2t, b2r):
    n_hid = w1c.shape[1]
    pair_spec = pl.BlockSpec((_HB, 128), lambda i: (i, 0))
    full = lambda shape: pl.BlockSpec(shape, lambda i: (0, 0))
    return pl.pallas_call(
        _mlp_body,
        grid=(_B // (2 * _HB),),
        in_specs=[
            pair_spec, pair_spec, pair_spec,
            pl.BlockSpec((_HB, 16), lambda i: (i, 0)),
            full((_D, n_hid)), full((_D, n_hid)), full((_D, n_hid)),
            full((16, n_hid)), full((16, n_hid)), full((1, n_hid)),
            full((n_hid, _D)), full((1, _D)),
        ],
        out_specs=pair_spec,
        out_shape=jax.ShapeDtypeStruct((_B // 2, 128), jnp.float32),
    )(c2, s2, i2, sc2, w1c, w1s, w1i, wse, wso, b1r, w2t, b2r)


def kernel(category, sub_category, industry, average_score, client_feedback,
           total_awards_and_tips, cat_table, sub_table, ind_table,
           W1, b1, W2, b2):
    sub2 = _depad(sub_table.T)
    sub_flat = sub2.reshape(2 * sub2.shape[0], _D)

    cat_i2 = category.reshape(_B // _CHUNK, _CHUNK)
    sub_i2 = _pair_index(sub_category).reshape(_B // _CHUNK, _CHUNK)
    ind_i2 = industry.reshape(_B // _CHUNK, _CHUNK)

    cat_e, sub_e, ind_e = _sc_gather(
        cat_table, sub_flat, ind_table, cat_i2, sub_i2, ind_i2)
    c2 = cat_e.reshape(_B // 2, 128)
    s2 = sub_e.reshape(_B // 2, 128)
    i2 = ind_e.reshape(_B // 2, 128)

    w1t = W1.T
    w1c = w1t[0:_D]
    w1s = w1t[_D:2 * _D]
    w1i = w1t[2 * _D:3 * _D]
    nsc = w1t.shape[0] - 3 * _D
    wsc8 = jnp.concatenate(
        [w1t[3 * _D:], jnp.zeros((8 - nsc, w1t.shape[1]), jnp.float32)], axis=0)
    z8 = jnp.zeros_like(wsc8)
    wse = jnp.concatenate([wsc8, z8], axis=0)
    wso = jnp.concatenate([z8, wsc8], axis=0)
    a2 = average_score.reshape(_B // 2, 2)
    f2 = client_feedback.reshape(_B // 2, 2)
    t2 = total_awards_and_tips.reshape(_B // 2, 2)
    z5 = jnp.zeros((_B // 2, 5), jnp.float32)
    sc2 = jnp.concatenate(
        [a2[:, 0:1], f2[:, 0:1], t2[:, 0:1], z5,
         a2[:, 1:2], f2[:, 1:2], t2[:, 1:2], z5], axis=1)

    o2 = _tc_mlp(c2, s2, i2, sc2, w1c, w1s, w1i, wse, wso,
                 b1.reshape(1, -1), W2.T, b2.reshape(1, -1))
    return o2.reshape(_B, _D)


# speedup vs baseline: 1.9644x; 1.1199x over previous
import functools

import jax
import jax.numpy as jnp
from jax import lax
from jax.experimental import pallas as pl
from jax.experimental.pallas import tpu as pltpu
from jax.experimental.pallas import tpu_sc as plsc

_B = 16384
_D = 64
_NC, _NS = 2, 16
_NW = _NC * _NS
_BPW = _B // _NW
_CHUNK = 128
_NCH = _BPW // _CHUNK

_CB = 8192
_HB = 512


def _depad_body(t_ref, o_ref):
    t = t_ref[:]
    tt = jnp.transpose(t)
    t3 = tt.reshape(_CB // 16, 16, 64)
    c = jnp.concatenate([t3[:, 0:8, :], t3[:, 8:16, :]], axis=2)
    o_ref[:] = c.reshape(_CB // 2, 128)


@jax.jit
def _depad(t_t):
    n = t_t.shape[1]
    out_rows = pl.cdiv(n, 16) * 8
    return pl.pallas_call(
        _depad_body, grid=(pl.cdiv(n, _CB),),
        in_specs=[pl.BlockSpec((64, _CB), lambda i: (0, i))],
        out_specs=pl.BlockSpec((_CB // 2, 128), lambda i: (i, 0)),
        out_shape=jax.ShapeDtypeStruct((out_rows, 128), jnp.float32),
    )(t_t)


def _pair_index(idx):
    i = idx.astype(jnp.int32)
    return (i & ~jnp.int32(15)) + 2 * (i & 7) + ((i >> 3) & 1)


def _gather_body(cat_t, sub_t, ind_t, cat_i, sub_i, ind_i,
                 out_c, out_s, out_i,
                 idx, rows_c, rows_s, rows_i, sem):
    wid = lax.axis_index("s") * _NC + lax.axis_index("c")
    row0 = wid * _NCH

    pltpu.sync_copy(cat_i.at[pl.ds(row0, _NCH)], idx.at[0])
    pltpu.sync_copy(sub_i.at[pl.ds(row0, _NCH)], idx.at[1])
    pltpu.sync_copy(ind_i.at[pl.ds(row0, _NCH)], idx.at[2])

    copies = []
    for t, (tab, rows) in enumerate(
            ((cat_t, rows_c), (sub_t, rows_s), (ind_t, rows_i))):
        for j in range(_NCH):
            copies.append(pltpu.async_copy(
                tab.at[idx.at[t, j]],
                rows.at[pl.ds(j * _CHUNK, _CHUNK)], sem))
    for c in copies:
        c.wait()

    base = wid * _BPW
    pltpu.sync_copy(rows_c, out_c.at[pl.ds(base, _BPW)])
    pltpu.sync_copy(rows_s, out_s.at[pl.ds(base, _BPW)])
    pltpu.sync_copy(rows_i, out_i.at[pl.ds(base, _BPW)])


@jax.jit
def _sc_gather(cat_table, sub_table, ind_table, cat_i2, sub_i2, ind_i2):
    mesh = plsc.VectorSubcoreMesh(
        core_axis_name="c", subcore_axis_name="s",
        num_cores=_NC, num_subcores=_NS)
    f = pl.kernel(
        _gather_body,
        out_type=[jax.ShapeDtypeStruct((_B, _D), jnp.float32)] * 3,
        mesh=mesh,
        scratch_types=[
            pltpu.VMEM((3, _NCH, _CHUNK), jnp.int32),
            pltpu.VMEM((_BPW, _D), jnp.float32),
            pltpu.VMEM((_BPW, _D), jnp.float32),
            pltpu.VMEM((_BPW, _D), jnp.float32),
            pltpu.SemaphoreType.DMA,
        ],
        compiler_params=pltpu.CompilerParams(use_tc_tiling_on_sc=False),
    )
    return f(cat_table, sub_table, ind_table, cat_i2, sub_i2, ind_i2)


def _mlp_body(c2_ref, s2_ref, i2_ref, sc2_ref, w1c_ref, w1s_ref, w1i_ref,
              wsc_ref, b1_ref, w2t_ref, b2_ref, o_ref):
    def dot(a, b):
        return jnp.dot(a, b, preferred_element_type=jnp.float32)

    he = (dot(c2_ref[:, 0:64], w1c_ref[:]) + dot(s2_ref[:, 0:64], w1s_ref[:])
          + dot(i2_ref[:, 0:64], w1i_ref[:]) + dot(sc2_ref[:, 0:64], wsc_ref[:])
          + b1_ref[:])
    ho = (dot(c2_ref[:, 64:128], w1c_ref[:]) + dot(s2_ref[:, 64:128], w1s_ref[:])
          + dot(i2_ref[:, 64:128], w1i_ref[:]) + dot(sc2_ref[:, 64:128], wsc_ref[:])
          + b1_ref[:])
    he = jnp.maximum(he, 0.0)
    ho = jnp.maximum(ho, 0.0)
    o_ref[:] = jnp.concatenate(
        [dot(he, w2t_ref[:]) + b2_ref[:], dot(ho, w2t_ref[:]) + b2_ref[:]],
        axis=1)


@jax.jit
def _tc_mlp(c2, s2, i2, sc2, w1c, w1s, w1i, wsc, b1r, w2t, b2r):
    n_hid = w1c.shape[1]
    pair_spec = pl.BlockSpec((_HB, 128), lambda i: (i, 0))
    full = lambda shape: pl.BlockSpec(shape, lambda i: (0, 0))
    return pl.pallas_call(
        _mlp_body,
        grid=(_B // (2 * _HB),),
        in_specs=[
            pair_spec, pair_spec, pair_spec, pair_spec,
            full((_D, n_hid)), full((_D, n_hid)), full((_D, n_hid)),
            full((_D, n_hid)), full((1, n_hid)),
            full((n_hid, _D)), full((1, _D)),
        ],
        out_specs=pair_spec,
        out_shape=jax.ShapeDtypeStruct((_B // 2, 128), jnp.float32),
    )(c2, s2, i2, sc2, w1c, w1s, w1i, wsc, b1r, w2t, b2r)


def kernel(category, sub_category, industry, average_score, client_feedback,
           total_awards_and_tips, cat_table, sub_table, ind_table,
           W1, b1, W2, b2):
    sub2 = _depad(sub_table.T)
    sub_flat = sub2.reshape(2 * sub2.shape[0], _D)

    cat_i2 = category.reshape(_B // _CHUNK, _CHUNK)
    sub_i2 = _pair_index(sub_category).reshape(_B // _CHUNK, _CHUNK)
    ind_i2 = industry.reshape(_B // _CHUNK, _CHUNK)

    cat_e, sub_e, ind_e = _sc_gather(
        cat_table, sub_flat, ind_table, cat_i2, sub_i2, ind_i2)
    c2 = cat_e.reshape(_B // 2, 128)
    s2 = sub_e.reshape(_B // 2, 128)
    i2 = ind_e.reshape(_B // 2, 128)

    w1t = W1.T
    w1c = w1t[0:_D]
    w1s = w1t[_D:2 * _D]
    w1i = w1t[2 * _D:3 * _D]
    nsc = w1t.shape[0] - 3 * _D
    wsc = jnp.concatenate(
        [w1t[3 * _D:], jnp.zeros((_D - nsc, w1t.shape[1]), jnp.float32)], axis=0)
    s64 = jnp.concatenate(
        [average_score, client_feedback, total_awards_and_tips,
         jnp.zeros((_B, _D - 3), jnp.float32)], axis=1)
    sc2 = s64.reshape(_B // 2, 128)

    o2 = _tc_mlp(c2, s2, i2, sc2, w1c, w1s, w1i, wsc,
                 b1.reshape(1, -1), W2.T, b2.reshape(1, -1))
    return o2.reshape(_B, _D)
